# baseline (speedup 1.0000x reference)
import jax
import jax.numpy as jnp
from jax import lax
from jax.experimental import pallas as pl
from jax.experimental.pallas import tpu as pltpu
from jax.experimental.pallas import tpu_sc as plsc

_N = 32768
_Q = 4096
_K = 25
_C = 32
_NC = _N // _C
_QB = 128
_NCORES = 2
_NSUB = 16
_NW = _NCORES * _NSUB
_QW = _Q // _NW
_SB = 8
_QSB = _QW // _SB
_KP = 32
_BIG = float(1e30)


def _round_bf16(a):
    return a.astype(jnp.bfloat16).astype(jnp.float32)


def _chunk_topk_body(x_ref, y_ref, xt_ref, yt_ref, out_ref, cm_ref):
    xq = x_ref[...]
    yq = y_ref[...]
    qq = xq * xq + yq * yq
    xqb = _round_bf16(xq)
    yqb = _round_bf16(yq)
    cm = None
    for r in range(_C):
        xr = xt_ref[r:r + 1, :]
        yr = yt_ref[r:r + 1, :]
        s = xqb * _round_bf16(xr) + yqb * _round_bf16(yr)
        d = (qq - 2.0 * s) + (xr * xr + yr * yr)
        cm = d if cm is None else jnp.minimum(cm, d)
    cm_ref[...] = cm
    iota = lax.broadcasted_iota(jnp.int32, (_QB, _NC), 1)
    for t in range(_K):
        cmv = cm_ref[...]
        m = jnp.min(cmv, axis=1, keepdims=True)
        idx = jnp.min(jnp.where(cmv == m, iota, jnp.int32(2 ** 30)),
                      axis=1, keepdims=True)
        out_ref[:, t:t + 1] = idx
        cm_ref[...] = jnp.where(iota == idx, _BIG, cmv)


def _gather_body(idx_hbm, table_hbm, out_hbm, idx_v, rows_v, sem):
    w = lax.axis_index("s") * _NCORES + lax.axis_index("c")
    for b in range(_SB):
        pltpu.sync_copy(idx_hbm.at[w, b], idx_v)
        cps = []
        for i in range(_QSB):
            cps.append(pltpu.async_copy(
                table_hbm.at[idx_v.at[i, pl.ds(0, _K)]], rows_v.at[i], sem))
        for cp in cps:
            cp.wait()
        pltpu.sync_copy(rows_v, out_hbm.at[w, b])


def _combine_body(x_ref, y_ref, xn_ref, yn_ref, hn_ref, un_ref, nid_ref,
                  o_ref):
    xq = x_ref[...]
    yq = y_ref[...]
    xnv = xn_ref[...]
    ynv = yn_ref[...]
    qq = xq * xq + yq * yq
    s = (_round_bf16(xq) * _round_bf16(xnv)
         + _round_bf16(yq) * _round_bf16(ynv))
    d2 = (qq - 2.0 * s) + (xnv * xnv + ynv * ynv)
    nid = nid_ref[...]
    keep = jnp.zeros(d2.shape, dtype=jnp.bool_)
    v = d2
    for _ in range(_K):
        m = jnp.min(v, axis=1, keepdims=True)
        sel = v == m
        p = jnp.min(jnp.where(sel, nid, jnp.int32(2 ** 30)),
                    axis=1, keepdims=True)
        chosen = sel & (nid == p)
        keep = keep | chosen
        v = jnp.where(chosen, _BIG, v)
    hv = hn_ref[...]
    dx = xq - xnv
    dy = yq - ynv
    xh = dx / hv
    yh = dy / hv
    e = jnp.exp(-(xh * xh + yh * yh))
    kern = jnp.where(keep, e, jnp.float32(0.0))
    dnr = jnp.sum(kern, axis=1, keepdims=True)
    nr = jnp.sum(un_ref[...] * kern, axis=1, keepdims=True)
    o_ref[...] = nr / dnr


def _stage_a(x2, y2, xt, yt, interpret=False):
    return pl.pallas_call(
        _chunk_topk_body,
        grid=(_Q // _QB,),
        in_specs=[
            pl.BlockSpec((_QB, 1), lambda i: (i, 0)),
            pl.BlockSpec((_QB, 1), lambda i: (i, 0)),
            pl.BlockSpec((_C, _NC), lambda i: (0, 0)),
            pl.BlockSpec((_C, _NC), lambda i: (0, 0)),
        ],
        out_specs=pl.BlockSpec((_QB, _K), lambda i: (i, 0)),
        out_shape=jax.ShapeDtypeStruct((_Q, _K), jnp.int32),
        scratch_shapes=[pltpu.VMEM((_QB, _NC), jnp.float32)],
        interpret=interpret,
    )(x2, y2, xt, yt)


def _stage_b(idx4, table):
    return pl.kernel(
        _gather_body,
        out_type=jax.ShapeDtypeStruct((_NW, _SB, _QSB, _K, 4 * _C),
                                      jnp.float32),
        mesh=plsc.VectorSubcoreMesh(core_axis_name="c", subcore_axis_name="s",
                                    num_cores=_NCORES, num_subcores=_NSUB),
        scratch_types=[
            pltpu.VMEM((_QSB, _KP), jnp.int32),
            pltpu.VMEM((_QSB, _K, 4 * _C), jnp.float32),
            pltpu.SemaphoreType.DMA,
        ],
    )(idx4, table)


def _stage_c(x2, y2, xn, yn, hn, un, nid, interpret=False):
    return pl.pallas_call(
        _combine_body,
        grid=(_Q // _QB,),
        in_specs=[
            pl.BlockSpec((_QB, 1), lambda i: (i, 0)),
            pl.BlockSpec((_QB, 1), lambda i: (i, 0)),
        ] + [pl.BlockSpec((_QB, _K * _C), lambda i: (i, 0))] * 5,
        out_specs=pl.BlockSpec((_QB, 1), lambda i: (i, 0)),
        out_shape=jax.ShapeDtypeStruct((_Q, 1), jnp.float32),
        interpret=interpret,
    )(x2, y2, xn, yn, hn, un, nid)


def kernel(x, y, nodes, h, u):
    xs = nodes[:, 0].reshape(_NC, _C)
    ys = nodes[:, 1].reshape(_NC, _C)
    x2 = x.reshape(_Q, 1)
    y2 = y.reshape(_Q, 1)

    cidx = _stage_a(x2, y2, xs.T, ys.T)

    idx4 = jnp.pad(cidx, ((0, 0), (0, _KP - _K))).reshape(
        _NW, _SB, _QSB, _KP)
    table = jnp.concatenate(
        [xs, ys, h.reshape(_NC, _C), u.reshape(_NC, _C)], axis=1)

    g3 = _stage_b(idx4, table).reshape(_Q, _K, 4 * _C)

    xn = g3[:, :, 0:_C].reshape(_Q, _K * _C)
    yn = g3[:, :, _C:2 * _C].reshape(_Q, _K * _C)
    hn = g3[:, :, 2 * _C:3 * _C].reshape(_Q, _K * _C)
    un = g3[:, :, 3 * _C:4 * _C].reshape(_Q, _K * _C)
    nid = (cidx[:, :, None] * _C
           + jnp.arange(_C, dtype=jnp.int32)[None, None, :]
           ).reshape(_Q, _K * _C)

    o2 = _stage_c(x2, y2, xn, yn, hn, un, nid)
    return o2.reshape(_Q)


# speedup vs baseline: 1.2070x; 1.2070x over previous
import jax
import jax.numpy as jnp
from jax import lax
from jax.experimental import pallas as pl
from jax.experimental.pallas import tpu as pltpu
from jax.experimental.pallas import tpu_sc as plsc

_N = 32768
_Q = 4096
_K = 25
_C = 32
_NC = _N // _C
_QB = 128
_NCORES = 2
_NSUB = 16
_NW = _NCORES * _NSUB
_QW = _Q // _NW
_RW = _QW * _K // 128
_SB = 5
_RSB = _RW // _SB
_DR = 2048
_BIG = float(1e30)


def _round_bf16(a):
    return a.astype(jnp.bfloat16).astype(jnp.float32)


def _chunk_topk_body(x_ref, y_ref, xt_ref, yt_ref, out_ref, cm_ref):
    xq = x_ref[...]
    yq = y_ref[...]
    qq = xq * xq + yq * yq
    xqb = _round_bf16(xq)
    yqb = _round_bf16(yq)
    cm = None
    for r in range(_C):
        xr = xt_ref[r:r + 1, :]
        yr = yt_ref[r:r + 1, :]
        s = xqb * _round_bf16(xr) + yqb * _round_bf16(yr)
        d = (qq - 2.0 * s) + (xr * xr + yr * yr)
        cm = d if cm is None else jnp.minimum(cm, d)
    cm_ref[...] = cm
    iota = lax.broadcasted_iota(jnp.int32, (_QB, _NC), 1)
    for t in range(_K):
        cmv = cm_ref[...]
        m = jnp.min(cmv, axis=1, keepdims=True)
        idx = jnp.min(jnp.where(cmv == m, iota, jnp.int32(2 ** 30)),
                      axis=1, keepdims=True)
        out_ref[:, t:t + 1] = idx
        cm_ref[...] = jnp.where(iota == idx, _BIG, cmv)


def _gather_body(idx_hbm, table_hbm, out_hbm, idx_v, rows_v, sem):
    w = lax.axis_index("s") * _NCORES + lax.axis_index("c")
    pltpu.sync_copy(idx_hbm.at[w], idx_v)
    for b in range(_SB):
        cps = []
        for r in range(_RSB):
            cps.append(pltpu.async_copy(
                table_hbm.at[idx_v.at[b * _RSB + r]], rows_v.at[r], sem))
        for cp in cps:
            cp.wait()
        pltpu.sync_copy(rows_v, out_hbm.at[w, b])


def _combine_body(x_ref, y_ref, xn_ref, yn_ref, hn_ref, un_ref, o_ref):
    xq = x_ref[...]
    yq = y_ref[...]
    xnv = xn_ref[...]
    ynv = yn_ref[...]
    qq = xq * xq + yq * yq
    s = (_round_bf16(xq) * _round_bf16(xnv)
         + _round_bf16(yq) * _round_bf16(ynv))
    d2 = (qq - 2.0 * s) + (xnv * xnv + ynv * ynv)
    nid = lax.broadcasted_iota(jnp.int32, d2.shape, 1)
    keep = jnp.zeros(d2.shape, dtype=jnp.bool_)
    v = d2
    for _ in range(_K):
        m = jnp.min(v, axis=1, keepdims=True)
        sel = v == m
        p = jnp.min(jnp.where(sel, nid, jnp.int32(2 ** 30)),
                    axis=1, keepdims=True)
        chosen = sel & (nid == p)
        keep = keep | chosen
        v = jnp.where(chosen, _BIG, v)
    hv = hn_ref[...]
    dx = xq - xnv
    dy = yq - ynv
    xh = dx / hv
    yh = dy / hv
    e = jnp.exp(-(xh * xh + yh * yh))
    kern = jnp.where(keep, e, jnp.float32(0.0))
    dnr = jnp.sum(kern, axis=1, keepdims=True)
    nr = jnp.sum(un_ref[...] * kern, axis=1, keepdims=True)
    o_ref[...] = nr / dnr


def _stage_a(x2, y2, xt, yt, interpret=False):
    return pl.pallas_call(
        _chunk_topk_body,
        grid=(_Q // _QB,),
        in_specs=[
            pl.BlockSpec((_QB, 1), lambda i: (i, 0)),
            pl.BlockSpec((_QB, 1), lambda i: (i, 0)),
            pl.BlockSpec((_C, _NC), lambda i: (0, 0)),
            pl.BlockSpec((_C, _NC), lambda i: (0, 0)),
        ],
        out_specs=pl.BlockSpec((_QB, _K), lambda i: (i, 0)),
        out_shape=jax.ShapeDtypeStruct((_Q, _K), jnp.int32),
        scratch_shapes=[pltpu.VMEM((_QB, _NC), jnp.float32)],
        interpret=interpret,
    )(x2, y2, xt, yt)


def _stage_b(idx3, table):
    return pl.kernel(
        _gather_body,
        out_type=jax.ShapeDtypeStruct((_NW, _SB, _RSB, 128, 4 * _C),
                                      jnp.float32),
        mesh=plsc.VectorSubcoreMesh(core_axis_name="c", subcore_axis_name="s",
                                    num_cores=_NCORES, num_subcores=_NSUB),
        scratch_types=[
            pltpu.VMEM((_RW, 128), jnp.int32),
            pltpu.VMEM((_RSB, 128, 4 * _C), jnp.float32),
            pltpu.SemaphoreType.DMA,
        ],
    )(idx3, table)


def _deint_body(g_ref, x_ref, y_ref, h_ref, u_ref):
    v = g_ref[...]
    x_ref[...] = v[:, 0:_C]
    y_ref[...] = v[:, _C:2 * _C]
    h_ref[...] = v[:, 2 * _C:3 * _C]
    u_ref[...] = v[:, 3 * _C:4 * _C]


def _stage_d(g2, interpret=False):
    out = jax.ShapeDtypeStruct((_Q * _K, _C), jnp.float32)
    return pl.pallas_call(
        _deint_body,
        grid=(_Q * _K // _DR,),
        in_specs=[pl.BlockSpec((_DR, 4 * _C), lambda i: (i, 0))],
        out_specs=[pl.BlockSpec((_DR, _C), lambda i: (i, 0))] * 4,
        out_shape=(out,) * 4,
        interpret=interpret,
    )(g2)


def _stage_c(x2, y2, xn, yn, hn, un, interpret=False):
    return pl.pallas_call(
        _combine_body,
        grid=(_Q // _QB,),
        in_specs=[
            pl.BlockSpec((_QB, 1), lambda i: (i, 0)),
            pl.BlockSpec((_QB, 1), lambda i: (i, 0)),
        ] + [pl.BlockSpec((_QB, _K * _C), lambda i: (i, 0))] * 4,
        out_specs=pl.BlockSpec((_QB, 1), lambda i: (i, 0)),
        out_shape=jax.ShapeDtypeStruct((_Q, 1), jnp.float32),
        interpret=interpret,
    )(x2, y2, xn, yn, hn, un)


def kernel(x, y, nodes, h, u):
    xs = nodes[:, 0].reshape(_NC, _C)
    ys = nodes[:, 1].reshape(_NC, _C)
    x2 = x.reshape(_Q, 1)
    y2 = y.reshape(_Q, 1)

    cidx = _stage_a(x2, y2, xs.T, ys.T)
    cidx = lax.sort(cidx, dimension=1)

    idx3 = cidx.reshape(_NW, _RW, 128)
    table = jnp.concatenate(
        [xs, ys, h.reshape(_NC, _C), u.reshape(_NC, _C)], axis=1)

    g2 = _stage_b(idx3, table).reshape(_Q * _K, 4 * _C)

    dx_, dy_, dh_, du_ = _stage_d(g2)
    xn = dx_.reshape(_Q, _K * _C)
    yn = dy_.reshape(_Q, _K * _C)
    hn = dh_.reshape(_Q, _K * _C)
    un = du_.reshape(_Q, _K * _C)

    o2 = _stage_c(x2, y2, xn, yn, hn, un)
    return o2.reshape(_Q)


# speedup vs baseline: 1.3997x; 1.1596x over previous
import jax
import jax.numpy as jnp
from jax import lax
from jax.experimental import pallas as pl
from jax.experimental.pallas import tpu as pltpu
from jax.experimental.pallas import tpu_sc as plsc

_N = 32768
_Q = 4096
_K = 25
_C = 32
_NC = _N // _C
_QB = 128
_NCORES = 2
_NSUB = 16
_NW = _NCORES * _NSUB
_QW = _Q // _NW
_RW = _QW * _K // 128
_SB = 5
_RSB = _RW // _SB
_DR = 2048
_BIG = float(1e30)


def _round_bf16(a):
    return a.astype(jnp.bfloat16).astype(jnp.float32)


def _chunk_topk_body(x_ref, y_ref, xt_ref, yt_ref, out_ref, cm_ref):
    xq = x_ref[...]
    yq = y_ref[...]
    qq = xq * xq + yq * yq
    xqb = _round_bf16(xq)
    yqb = _round_bf16(yq)
    cm = None
    for r in range(_C):
        xr = xt_ref[r:r + 1, :]
        yr = yt_ref[r:r + 1, :]
        s = xqb * _round_bf16(xr) + yqb * _round_bf16(yr)
        d = (qq - 2.0 * s) + (xr * xr + yr * yr)
        cm = d if cm is None else jnp.minimum(cm, d)
    cm_ref[...] = cm
    iota = lax.broadcasted_iota(jnp.int32, (_QB, _NC), 1).astype(jnp.float32)
    for t in range(_K):
        cmv = cm_ref[...]
        m = jnp.min(cmv, axis=1, keepdims=True)
        idxf = jnp.min(jnp.where(cmv == m, iota, jnp.float32(1e9)),
                       axis=1, keepdims=True)
        out_ref[:, t:t + 1] = idxf.astype(jnp.int32)
        cm_ref[...] = jnp.where(iota == idxf, _BIG, cmv)


def _gather_body(idx_hbm, table_hbm, out_hbm, idx_v, rows_v, sem):
    w = lax.axis_index("s") * _NCORES + lax.axis_index("c")
    pltpu.sync_copy(idx_hbm.at[w], idx_v)
    for b in range(_SB):
        cps = []
        for r in range(_RSB):
            cps.append(pltpu.async_copy(
                table_hbm.at[idx_v.at[b * _RSB + r]], rows_v.at[r], sem))
        for cp in cps:
            cp.wait()
        pltpu.sync_copy(rows_v, out_hbm.at[w, b])


def _combine_body(x_ref, y_ref, xn_ref, yn_ref, hn_ref, un_ref, o_ref):
    xq = x_ref[...]
    yq = y_ref[...]
    xnv = xn_ref[...]
    ynv = yn_ref[...]
    qq = xq * xq + yq * yq
    s = (_round_bf16(xq) * _round_bf16(xnv)
         + _round_bf16(yq) * _round_bf16(ynv))
    d2 = (qq - 2.0 * s) + (xnv * xnv + ynv * ynv)
    nid = lax.broadcasted_iota(jnp.int32, d2.shape, 1).astype(jnp.float32)
    keep = jnp.zeros(d2.shape, dtype=jnp.bool_)
    v = d2
    for _ in range(_K):
        m = jnp.min(v, axis=1, keepdims=True)
        sel = v == m
        p = jnp.min(jnp.where(sel, nid, jnp.float32(1e9)),
                    axis=1, keepdims=True)
        chosen = sel & (nid == p)
        keep = keep | chosen
        v = jnp.where(chosen, _BIG, v)
    hv = hn_ref[...]
    dx = xq - xnv
    dy = yq - ynv
    xh = dx / hv
    yh = dy / hv
    e = jnp.exp(-(xh * xh + yh * yh))
    kern = jnp.where(keep, e, jnp.float32(0.0))
    dnr = jnp.sum(kern, axis=1, keepdims=True)
    nr = jnp.sum(un_ref[...] * kern, axis=1, keepdims=True)
    o_ref[...] = nr / dnr


def _stage_a(x2, y2, xt, yt, interpret=False):
    return pl.pallas_call(
        _chunk_topk_body,
        grid=(_Q // _QB,),
        in_specs=[
            pl.BlockSpec((_QB, 1), lambda i: (i, 0)),
            pl.BlockSpec((_QB, 1), lambda i: (i, 0)),
            pl.BlockSpec((_C, _NC), lambda i: (0, 0)),
            pl.BlockSpec((_C, _NC), lambda i: (0, 0)),
        ],
        out_specs=pl.BlockSpec((_QB, _K), lambda i: (i, 0)),
        out_shape=jax.ShapeDtypeStruct((_Q, _K), jnp.int32),
        scratch_shapes=[pltpu.VMEM((_QB, _NC), jnp.float32)],
        interpret=interpret,
    )(x2, y2, xt, yt)


def _stage_b(idx3, table):
    return pl.kernel(
        _gather_body,
        out_type=jax.ShapeDtypeStruct((_NW, _SB, _RSB, 128, 4 * _C),
                                      jnp.float32),
        mesh=plsc.VectorSubcoreMesh(core_axis_name="c", subcore_axis_name="s",
                                    num_cores=_NCORES, num_subcores=_NSUB),
        scratch_types=[
            pltpu.VMEM((_RW, 128), jnp.int32),
            pltpu.VMEM((_RSB, 128, 4 * _C), jnp.float32),
            pltpu.SemaphoreType.DMA,
        ],
    )(idx3, table)


def _deint_body(g_ref, x_ref, y_ref, h_ref, u_ref):
    v = g_ref[...]
    x_ref[...] = v[:, 0:_C]
    y_ref[...] = v[:, _C:2 * _C]
    h_ref[...] = v[:, 2 * _C:3 * _C]
    u_ref[...] = v[:, 3 * _C:4 * _C]


def _stage_d(g2, interpret=False):
    out = jax.ShapeDtypeStruct((_Q * _K, _C), jnp.float32)
    return pl.pallas_call(
        _deint_body,
        grid=(_Q * _K // _DR,),
        in_specs=[pl.BlockSpec((_DR, 4 * _C), lambda i: (i, 0))],
        out_specs=[pl.BlockSpec((_DR, _C), lambda i: (i, 0))] * 4,
        out_shape=(out,) * 4,
        interpret=interpret,
    )(g2)


def _stage_c(x2, y2, xn, yn, hn, un, interpret=False):
    return pl.pallas_call(
        _combine_body,
        grid=(_Q // _QB,),
        in_specs=[
            pl.BlockSpec((_QB, 1), lambda i: (i, 0)),
            pl.BlockSpec((_QB, 1), lambda i: (i, 0)),
        ] + [pl.BlockSpec((_QB, _K * _C), lambda i: (i, 0))] * 4,
        out_specs=pl.BlockSpec((_QB, 1), lambda i: (i, 0)),
        out_shape=jax.ShapeDtypeStruct((_Q, 1), jnp.float32),
        interpret=interpret,
    )(x2, y2, xn, yn, hn, un)


def kernel(x, y, nodes, h, u):
    xs = nodes[:, 0].reshape(_NC, _C)
    ys = nodes[:, 1].reshape(_NC, _C)
    x2 = x.reshape(_Q, 1)
    y2 = y.reshape(_Q, 1)

    cidx = _stage_a(x2, y2, xs.T, ys.T)
    cidx = lax.sort(cidx, dimension=1)

    idx3 = cidx.reshape(_NW, _RW, 128)
    table = jnp.concatenate(
        [xs, ys, h.reshape(_NC, _C), u.reshape(_NC, _C)], axis=1)

    g2 = _stage_b(idx3, table).reshape(_Q * _K, 4 * _C)

    dx_, dy_, dh_, du_ = _stage_d(g2)
    xn = dx_.reshape(_Q, _K * _C)
    yn = dy_.reshape(_Q, _K * _C)
    hn = dh_.reshape(_Q, _K * _C)
    un = du_.reshape(_Q, _K * _C)

    o2 = _stage_c(x2, y2, xn, yn, hn, un)
    return o2.reshape(_Q)


# speedup vs baseline: 1.4410x; 1.0295x over previous
import jax
import jax.numpy as jnp
from jax import lax
from jax.experimental import pallas as pl
from jax.experimental.pallas import tpu as pltpu
from jax.experimental.pallas import tpu_sc as plsc

_N = 32768
_Q = 4096
_K = 25
_C = 32
_NC = _N // _C
_QB = 128
_NCORES = 2
_NSUB = 16
_NW = _NCORES * _NSUB
_QW = _Q // _NW
_RW = _QW * _K // 128
_SB = 5
_RSB = _RW // _SB
_DR = 2048
_BIG = float(1e30)


def _round_bf16(a):
    return a.astype(jnp.bfloat16).astype(jnp.float32)


def _chunk_topk_body(x_ref, y_ref, xt_ref, yt_ref, out_ref, cm_ref):
    xq = x_ref[...]
    yq = y_ref[...]
    qq = xq * xq + yq * yq
    xqb = _round_bf16(xq)
    yqb = _round_bf16(yq)
    cm = None
    for r in range(_C):
        xr = xt_ref[r:r + 1, :]
        yr = yt_ref[r:r + 1, :]
        s = xqb * _round_bf16(xr) + yqb * _round_bf16(yr)
        d = (qq - 2.0 * s) + (xr * xr + yr * yr)
        cm = d if cm is None else jnp.minimum(cm, d)
    cm_ref[...] = cm
    iota = lax.broadcasted_iota(jnp.int32, (_QB, _NC), 1).astype(jnp.float32)
    for t in range(_K):
        cmv = cm_ref[...]
        m = jnp.min(cmv, axis=1, keepdims=True)
        idxf = jnp.min(jnp.where(cmv == m, iota, jnp.float32(1e9)),
                       axis=1, keepdims=True)
        out_ref[:, t:t + 1] = idxf.astype(jnp.int32)
        cm_ref[...] = jnp.where(iota == idxf, _BIG, cmv)


def _gather_body(idx_hbm, table_hbm, out_hbm, idx_v, rows_v, sem):
    w = lax.axis_index("s") * _NCORES + lax.axis_index("c")
    pltpu.sync_copy(idx_hbm.at[w], idx_v)
    for b in range(_SB):
        cps = []
        for r in range(_RSB):
            cps.append(pltpu.async_copy(
                table_hbm.at[idx_v.at[b * _RSB + r]], rows_v.at[r], sem))
        for cp in cps:
            cp.wait()
        pltpu.sync_copy(rows_v, out_hbm.at[w, b])


def _combine_body(x_ref, y_ref, xn_ref, yn_ref, hn_ref, un_ref, o_ref):
    xq = x_ref[...]
    yq = y_ref[...]
    xnv = xn_ref[...]
    ynv = yn_ref[...]
    qq = xq * xq + yq * yq
    s = (_round_bf16(xq) * _round_bf16(xnv)
         + _round_bf16(yq) * _round_bf16(ynv))
    d2 = (qq - 2.0 * s) + (xnv * xnv + ynv * ynv)
    nid = lax.broadcasted_iota(jnp.int32, d2.shape, 1).astype(jnp.float32)
    keep = jnp.zeros(d2.shape, dtype=jnp.bool_)
    v = d2
    for _ in range(_K):
        m = jnp.min(v, axis=1, keepdims=True)
        sel = v == m
        p = jnp.min(jnp.where(sel, nid, jnp.float32(1e9)),
                    axis=1, keepdims=True)
        chosen = sel & (nid == p)
        keep = keep | chosen
        v = jnp.where(chosen, _BIG, v)
    hv = hn_ref[...]
    dx = xq - xnv
    dy = yq - ynv
    xh = dx / hv
    yh = dy / hv
    e = jnp.exp(-(xh * xh + yh * yh))
    kern = jnp.where(keep, e, jnp.float32(0.0))
    dnr = jnp.sum(kern, axis=1, keepdims=True)
    nr = jnp.sum(un_ref[...] * kern, axis=1, keepdims=True)
    o_ref[...] = nr / dnr


def _stage_a(x2, y2, xt, yt, interpret=False):
    qp = x2.shape[0]
    return pl.pallas_call(
        _chunk_topk_body,
        grid=(qp // _QB,),
        in_specs=[
            pl.BlockSpec((_QB, 1), lambda i: (i, 0)),
            pl.BlockSpec((_QB, 1), lambda i: (i, 0)),
            pl.BlockSpec((_C, _NC), lambda i: (0, 0)),
            pl.BlockSpec((_C, _NC), lambda i: (0, 0)),
        ],
        out_specs=pl.BlockSpec((_QB, _K), lambda i: (i, 0)),
        out_shape=jax.ShapeDtypeStruct((qp, _K), jnp.int32),
        scratch_shapes=[pltpu.VMEM((_QB, _NC), jnp.float32)],
        interpret=interpret,
    )(x2, y2, xt, yt)


def _stage_b(idx3, table):
    rl = idx3.shape[2]
    return pl.kernel(
        _gather_body,
        out_type=jax.ShapeDtypeStruct((_NW, _SB, _RSB, rl, 4 * _C),
                                      jnp.float32),
        mesh=plsc.VectorSubcoreMesh(core_axis_name="c", subcore_axis_name="s",
                                    num_cores=_NCORES, num_subcores=_NSUB),
        scratch_types=[
            pltpu.VMEM((_RW, rl), jnp.int32),
            pltpu.VMEM((_RSB, rl, 4 * _C), jnp.float32),
            pltpu.SemaphoreType.DMA,
        ],
    )(idx3, table)


def _deint_body(g_ref, x_ref, y_ref, h_ref, u_ref):
    v = g_ref[...]
    x_ref[...] = v[:, 0:_C]
    y_ref[...] = v[:, _C:2 * _C]
    h_ref[...] = v[:, 2 * _C:3 * _C]
    u_ref[...] = v[:, 3 * _C:4 * _C]


def _stage_d(g2, interpret=False):
    out = jax.ShapeDtypeStruct((g2.shape[0], _C), jnp.float32)
    return pl.pallas_call(
        _deint_body,
        grid=(g2.shape[0] // _DR,),
        in_specs=[pl.BlockSpec((_DR, 4 * _C), lambda i: (i, 0))],
        out_specs=[pl.BlockSpec((_DR, _C), lambda i: (i, 0))] * 4,
        out_shape=(out,) * 4,
        interpret=interpret,
    )(g2)


def _stage_c(x2, y2, xn, yn, hn, un, interpret=False):
    qp = x2.shape[0]
    return pl.pallas_call(
        _combine_body,
        grid=(qp // _QB,),
        in_specs=[
            pl.BlockSpec((_QB, 1), lambda i: (i, 0)),
            pl.BlockSpec((_QB, 1), lambda i: (i, 0)),
        ] + [pl.BlockSpec((_QB, _K * _C), lambda i: (i, 0))] * 4,
        out_specs=pl.BlockSpec((_QB, 1), lambda i: (i, 0)),
        out_shape=jax.ShapeDtypeStruct((qp, 1), jnp.float32),
        interpret=interpret,
    )(x2, y2, xn, yn, hn, un)


def kernel(x, y, nodes, h, u):
    xs = nodes[:, 0].reshape(_NC, _C)
    ys = nodes[:, 1].reshape(_NC, _C)
    x2 = x.reshape(_Q, 1)
    y2 = y.reshape(_Q, 1)

    table = jnp.concatenate(
        [xs, ys, h.reshape(_NC, _C), u.reshape(_NC, _C)], axis=1)

    outs = []
    qh = _Q // 2
    for p in range(2):
        sl = slice(p * qh, (p + 1) * qh)
        x2p, y2p = x2[sl], y2[sl]
        cidx = _stage_a(x2p, y2p, xs.T, ys.T)
        cidx = lax.sort(cidx, dimension=1)

        idx3 = cidx.reshape(_NW, _RW, qh * _K // (_NW * _RW))
        g2 = _stage_b(idx3, table).reshape(qh * _K, 4 * _C)

        dx_, dy_, dh_, du_ = _stage_d(g2)
        xn = dx_.reshape(qh, _K * _C)
        yn = dy_.reshape(qh, _K * _C)
        hn = dh_.reshape(qh, _K * _C)
        un = du_.reshape(qh, _K * _C)

        outs.append(_stage_c(x2p, y2p, xn, yn, hn, un))
    return jnp.concatenate(outs, axis=0).reshape(_Q)


# speedup vs baseline: 1.4414x; 1.0003x over previous
import jax
import jax.numpy as jnp
from jax import lax
from jax.experimental import pallas as pl
from jax.experimental.pallas import tpu as pltpu
from jax.experimental.pallas import tpu_sc as plsc

_N = 32768
_Q = 4096
_K = 25
_C = 32
_NC = _N // _C
_QB = 128
_NCORES = 2
_NSUB = 16
_NW = _NCORES * _NSUB
_QW = _Q // _NW
_RW = _QW * _K // 128
_SB = 5
_RSB = _RW // _SB
_DR = 2048
_BIG = float(1e30)


def _round_bf16(a):
    return a.astype(jnp.bfloat16).astype(jnp.float32)


def _chunk_topk_body(x_ref, y_ref, xt_ref, yt_ref, out_ref, cm_ref):
    xq = x_ref[...]
    yq = y_ref[...]
    qq = xq * xq + yq * yq
    xqb = _round_bf16(xq)
    yqb = _round_bf16(yq)
    cm = None
    for r in range(_C):
        xr = xt_ref[r:r + 1, :]
        yr = yt_ref[r:r + 1, :]
        s = xqb * _round_bf16(xr) + yqb * _round_bf16(yr)
        d = (qq - 2.0 * s) + (xr * xr + yr * yr)
        cm = d if cm is None else jnp.minimum(cm, d)
    cm_ref[...] = cm
    iota = lax.broadcasted_iota(jnp.int32, (_QB, _NC), 1).astype(jnp.float32)
    for t in range(_K):
        cmv = cm_ref[...]
        m = jnp.min(cmv, axis=1, keepdims=True)
        idxf = jnp.min(jnp.where(cmv == m, iota, jnp.float32(1e9)),
                       axis=1, keepdims=True)
        out_ref[:, t:t + 1] = idxf.astype(jnp.int32)
        cm_ref[...] = jnp.where(iota == idxf, _BIG, cmv)


def _gather_body(idx_hbm, table_hbm, out_hbm, idx_v, rows_v, sem):
    w = lax.axis_index("s") * _NCORES + lax.axis_index("c")
    pltpu.sync_copy(idx_hbm.at[w], idx_v)
    for b in range(_SB):
        cps = []
        for r in range(_RSB):
            cps.append(pltpu.async_copy(
                table_hbm.at[idx_v.at[b * _RSB + r]], rows_v.at[r], sem))
        for cp in cps:
            cp.wait()
        pltpu.sync_copy(rows_v, out_hbm.at[w, b])


def _combine_body(x_ref, y_ref, xn_ref, yn_ref, hn_ref, un_ref, o_ref):
    xq = x_ref[...]
    yq = y_ref[...]
    xnv = xn_ref[...]
    ynv = yn_ref[...]
    qq = xq * xq + yq * yq
    s = (_round_bf16(xq) * _round_bf16(xnv)
         + _round_bf16(yq) * _round_bf16(ynv))
    d2 = (qq - 2.0 * s) + (xnv * xnv + ynv * ynv)
    nid = lax.broadcasted_iota(jnp.int32, d2.shape, 1).astype(jnp.float32)
    keep = jnp.zeros(d2.shape, dtype=jnp.bool_)
    v = d2
    for _ in range(_K):
        m = jnp.min(v, axis=1, keepdims=True)
        sel = v == m
        p = jnp.min(jnp.where(sel, nid, jnp.float32(1e9)),
                    axis=1, keepdims=True)
        chosen = sel & (nid == p)
        keep = keep | chosen
        v = jnp.where(chosen, _BIG, v)
    hv = hn_ref[...]
    dx = xq - xnv
    dy = yq - ynv
    xh = dx / hv
    yh = dy / hv
    e = jnp.exp(-(xh * xh + yh * yh))
    kern = jnp.where(keep, e, jnp.float32(0.0))
    dnr = jnp.sum(kern, axis=1, keepdims=True)
    nr = jnp.sum(un_ref[...] * kern, axis=1, keepdims=True)
    o_ref[...] = nr / dnr


def _stage_a(x2, y2, xt, yt):
    qp = x2.shape[0]
    return pl.pallas_call(
        _chunk_topk_body,
        grid=(qp // _QB,),
        in_specs=[
            pl.BlockSpec((_QB, 1), lambda i: (i, 0)),
            pl.BlockSpec((_QB, 1), lambda i: (i, 0)),
            pl.BlockSpec((_C, _NC), lambda i: (0, 0)),
            pl.BlockSpec((_C, _NC), lambda i: (0, 0)),
        ],
        out_specs=pl.BlockSpec((_QB, _K), lambda i: (i, 0)),
        out_shape=jax.ShapeDtypeStruct((qp, _K), jnp.int32),
        scratch_shapes=[pltpu.VMEM((_QB, _NC), jnp.float32)],
    )(x2, y2, xt, yt)


def _stage_b(idx3, table):
    rl = idx3.shape[2]
    return pl.kernel(
        _gather_body,
        out_type=jax.ShapeDtypeStruct((_NW, _SB, _RSB, rl, 4 * _C),
                                      jnp.float32),
        mesh=plsc.VectorSubcoreMesh(core_axis_name="c", subcore_axis_name="s",
                                    num_cores=_NCORES, num_subcores=_NSUB),
        scratch_types=[
            pltpu.VMEM((_RW, rl), jnp.int32),
            pltpu.VMEM((_RSB, rl, 4 * _C), jnp.float32),
            pltpu.SemaphoreType.DMA,
        ],
    )(idx3, table)


def _deint_body(g_ref, x_ref, y_ref, h_ref, u_ref):
    v = g_ref[...]
    x_ref[...] = v[:, 0:_C]
    y_ref[...] = v[:, _C:2 * _C]
    h_ref[...] = v[:, 2 * _C:3 * _C]
    u_ref[...] = v[:, 3 * _C:4 * _C]


def _stage_d(g2):
    out = jax.ShapeDtypeStruct((g2.shape[0], _C), jnp.float32)
    return pl.pallas_call(
        _deint_body,
        grid=(g2.shape[0] // _DR,),
        in_specs=[pl.BlockSpec((_DR, 4 * _C), lambda i: (i, 0))],
        out_specs=[pl.BlockSpec((_DR, _C), lambda i: (i, 0))] * 4,
        out_shape=(out,) * 4,
    )(g2)


def _stage_c(x2, y2, xn, yn, hn, un):
    qp = x2.shape[0]
    return pl.pallas_call(
        _combine_body,
        grid=(qp // _QB,),
        in_specs=[
            pl.BlockSpec((_QB, 1), lambda i: (i, 0)),
            pl.BlockSpec((_QB, 1), lambda i: (i, 0)),
        ] + [pl.BlockSpec((_QB, _K * _C), lambda i: (i, 0))] * 4,
        out_specs=pl.BlockSpec((_QB, 1), lambda i: (i, 0)),
        out_shape=jax.ShapeDtypeStruct((qp, 1), jnp.float32),
    )(x2, y2, xn, yn, hn, un)


def kernel(x, y, nodes, h, u):
    xs = nodes[:, 0].reshape(_NC, _C)
    ys = nodes[:, 1].reshape(_NC, _C)
    x2 = x.reshape(_Q, 1)
    y2 = y.reshape(_Q, 1)

    table = jnp.concatenate(
        [xs, ys, h.reshape(_NC, _C), u.reshape(_NC, _C)], axis=1)

    outs = []
    qh = _Q // 2
    for p in range(2):
        sl = slice(p * qh, (p + 1) * qh)
        x2p, y2p = x2[sl], y2[sl]
        cidx = _stage_a(x2p, y2p, xs.T, ys.T)
        cidx = lax.sort(cidx, dimension=1)

        idx3 = cidx.reshape(_NW, _RW, qh * _K // (_NW * _RW))
        g2 = _stage_b(idx3, table).reshape(qh * _K, 4 * _C)

        dx_, dy_, dh_, du_ = _stage_d(g2)
        xn = dx_.reshape(qh, _K * _C)
        yn = dy_.reshape(qh, _K * _C)
        hn = dh_.reshape(qh, _K * _C)
        un = du_.reshape(qh, _K * _C)

        outs.append(_stage_c(x2p, y2p, xn, yn, hn, un))
    return jnp.concatenate(outs, axis=0).reshape(_Q)
